# initial kernel scaffold (unmeasured)
import jax
import jax.numpy as jnp
from jax import lax
from jax.experimental import pallas as pl
from jax.experimental.pallas import tpu as pltpu

N_DEV = 8
M = 4096
N = 8192
M_CHUNK = M // N_DEV


def kernel(x, w_mat, scale_x, scale_w):
    def body(x_ref, w_ref, sx_ref, sw_ref, out_ref, comm_ref,
             send_sems, recv_sems, credit_sem, copy_sem):
        my = lax.axis_index("i")
        left = lax.rem(my - 1 + N_DEV, N_DEV)
        right = lax.rem(my + 1, N_DEV)

        barrier_sem = pltpu.get_barrier_semaphore()
        for nbr in (left, right):
            pl.semaphore_signal(
                barrier_sem, inc=1,
                device_id=(nbr,), device_id_type=pl.DeviceIdType.MESH,
            )
        pl.semaphore_wait(barrier_sem, 2)

        w_bf = w_ref[:, :].astype(jnp.bfloat16)

        def partial_chunk(c):
            xb = x_ref[pl.ds(c * M_CHUNK, M_CHUNK), :].astype(jnp.bfloat16)
            return lax.dot_general(
                xb, w_bf, (((1,), (0,)), ((), ())),
                preferred_element_type=jnp.float32,
            )

        def credit_to_left():
            pl.semaphore_signal(
                credit_sem, inc=1,
                device_id=(left,), device_id_type=pl.DeviceIdType.MESH,
            )

        comm_ref[0, :, :] = partial_chunk(my)
        for s in range(N_DEV - 1):
            send_slot = s % 2
            recv_slot = (s + 1) % 2
            rdma = pltpu.make_async_remote_copy(
                src_ref=comm_ref.at[send_slot],
                dst_ref=comm_ref.at[recv_slot],
                send_sem=send_sems.at[send_slot],
                recv_sem=recv_sems.at[recv_slot],
                device_id=(right,),
                device_id_type=pl.DeviceIdType.MESH,
            )
            if s >= 1:
                pl.semaphore_wait(credit_sem, 1)
            rdma.start()
            rdma.wait()
            c = lax.rem(my - s - 1 + N_DEV, N_DEV)
            comm_ref[recv_slot, :, :] = comm_ref[recv_slot, :, :] + partial_chunk(c)
            credit_to_left()

        scale = sx_ref[0] * sw_ref[0]
        y = comm_ref[1, :, :] * scale
        yc = jnp.clip(y, -60.0, 60.0)
        comm_ref[1, :, :] = y * (1.0 / (1.0 + jnp.exp(-yc)))
        own = lax.rem(my + 1, N_DEV)
        cp = pltpu.make_async_copy(
            comm_ref.at[1],
            out_ref.at[pl.ds(own * M_CHUNK, M_CHUNK), :],
            copy_sem,
        )
        cp.start()
        cp.wait()

        for t in range(N_DEV - 1):
            send_slot = (t + 1) % 2
            recv_slot = t % 2
            rdma = pltpu.make_async_remote_copy(
                src_ref=comm_ref.at[send_slot],
                dst_ref=comm_ref.at[recv_slot],
                send_sem=send_sems.at[send_slot],
                recv_sem=recv_sems.at[recv_slot],
                device_id=(right,),
                device_id_type=pl.DeviceIdType.MESH,
            )
            pl.semaphore_wait(credit_sem, 1)
            rdma.start()
            rdma.wait()
            g = lax.rem(my - t + N_DEV, N_DEV)
            cp = pltpu.make_async_copy(
                comm_ref.at[recv_slot],
                out_ref.at[pl.ds(g * M_CHUNK, M_CHUNK), :],
                copy_sem,
            )
            cp.start()
            cp.wait()
            if t < N_DEV - 2:
                credit_to_left()

    return pl.pallas_call(
        body,
        out_shape=jax.ShapeDtypeStruct((M, N), jnp.float32),
        in_specs=[
            pl.BlockSpec(memory_space=pltpu.VMEM),
            pl.BlockSpec(memory_space=pltpu.VMEM),
            pl.BlockSpec(memory_space=pltpu.SMEM),
            pl.BlockSpec(memory_space=pltpu.SMEM),
        ],
        out_specs=pl.BlockSpec(memory_space=pltpu.ANY),
        scratch_shapes=[
            pltpu.VMEM((2, M_CHUNK, N), jnp.float32),
            pltpu.SemaphoreType.DMA((2,)),
            pltpu.SemaphoreType.DMA((2,)),
            pltpu.SemaphoreType.REGULAR,
            pltpu.SemaphoreType.DMA,
        ],
        compiler_params=pltpu.CompilerParams(collective_id=0),
    )(x, w_mat, scale_x, scale_w)


# baseline (device time: 2759981 ns/iter reference)
import jax
import jax.numpy as jnp
from jax import lax
from jax.experimental import pallas as pl
from jax.experimental.pallas import tpu as pltpu

N_DEV = 8
M = 4096
N = 8192
M_CHUNK = M // N_DEV


def kernel(x, w_mat, scale_x, scale_w):
    x = x.astype(jnp.bfloat16)
    w_mat = w_mat.astype(jnp.bfloat16)

    def body(x_ref, w_ref, sx_ref, sw_ref, out_ref, comm_ref,
             send_sems, recv_sems, credit_sem, copy_sem):
        my = lax.axis_index("i")
        left = lax.rem(my - 1 + N_DEV, N_DEV)
        right = lax.rem(my + 1, N_DEV)

        barrier_sem = pltpu.get_barrier_semaphore()
        for nbr in (left, right):
            pl.semaphore_signal(
                barrier_sem, inc=1,
                device_id=(nbr,), device_id_type=pl.DeviceIdType.MESH,
            )
        pl.semaphore_wait(barrier_sem, 2)

        def partial_chunk(c):
            xb = x_ref[pl.ds(c * M_CHUNK, M_CHUNK), :]
            return lax.dot_general(
                xb, w_ref[:, :], (((1,), (0,)), ((), ())),
                preferred_element_type=jnp.float32,
            )

        def credit_to_left():
            pl.semaphore_signal(
                credit_sem, inc=1,
                device_id=(left,), device_id_type=pl.DeviceIdType.MESH,
            )

        comm_ref[0, :, :] = partial_chunk(my)
        for s in range(N_DEV - 1):
            send_slot = s % 2
            recv_slot = (s + 1) % 2
            rdma = pltpu.make_async_remote_copy(
                src_ref=comm_ref.at[send_slot],
                dst_ref=comm_ref.at[recv_slot],
                send_sem=send_sems.at[send_slot],
                recv_sem=recv_sems.at[recv_slot],
                device_id=(right,),
                device_id_type=pl.DeviceIdType.MESH,
            )
            if s >= 1:
                pl.semaphore_wait(credit_sem, 1)
            rdma.start()
            rdma.wait()
            c = lax.rem(my - s - 1 + N_DEV, N_DEV)
            comm_ref[recv_slot, :, :] = comm_ref[recv_slot, :, :] + partial_chunk(c)
            credit_to_left()

        scale = sx_ref[0] * sw_ref[0]
        y = comm_ref[1, :, :] * scale
        yc = jnp.clip(y, -60.0, 60.0)
        comm_ref[1, :, :] = y * (1.0 / (1.0 + jnp.exp(-yc)))
        own = lax.rem(my + 1, N_DEV)
        cp = pltpu.make_async_copy(
            comm_ref.at[1],
            out_ref.at[pl.ds(own * M_CHUNK, M_CHUNK), :],
            copy_sem,
        )
        cp.start()
        cp.wait()

        for t in range(N_DEV - 1):
            send_slot = (t + 1) % 2
            recv_slot = t % 2
            rdma = pltpu.make_async_remote_copy(
                src_ref=comm_ref.at[send_slot],
                dst_ref=comm_ref.at[recv_slot],
                send_sem=send_sems.at[send_slot],
                recv_sem=recv_sems.at[recv_slot],
                device_id=(right,),
                device_id_type=pl.DeviceIdType.MESH,
            )
            pl.semaphore_wait(credit_sem, 1)
            rdma.start()
            rdma.wait()
            g = lax.rem(my - t + N_DEV, N_DEV)
            cp = pltpu.make_async_copy(
                comm_ref.at[recv_slot],
                out_ref.at[pl.ds(g * M_CHUNK, M_CHUNK), :],
                copy_sem,
            )
            cp.start()
            cp.wait()
            if t < N_DEV - 2:
                credit_to_left()

    return pl.pallas_call(
        body,
        out_shape=jax.ShapeDtypeStruct((M, N), jnp.float32),
        in_specs=[
            pl.BlockSpec(memory_space=pltpu.VMEM),
            pl.BlockSpec(memory_space=pltpu.VMEM),
            pl.BlockSpec(memory_space=pltpu.SMEM),
            pl.BlockSpec(memory_space=pltpu.SMEM),
        ],
        out_specs=pl.BlockSpec(memory_space=pl.ANY),
        scratch_shapes=[
            pltpu.VMEM((2, M_CHUNK, N), jnp.float32),
            pltpu.SemaphoreType.DMA((2,)),
            pltpu.SemaphoreType.DMA((2,)),
            pltpu.SemaphoreType.REGULAR,
            pltpu.SemaphoreType.DMA,
        ],
        compiler_params=pltpu.CompilerParams(
            collective_id=0,
            vmem_limit_bytes=100 * 1024 * 1024,
        ),
    )(x, w_mat, scale_x, scale_w)


# device time: 785601 ns/iter; 3.5132x vs baseline; 3.5132x over previous
import jax
import jax.numpy as jnp
from jax import lax
from jax.experimental import pallas as pl
from jax.experimental.pallas import tpu as pltpu

N_DEV = 8
M = 4096
N = 8192
M_CHUNK = M // N_DEV
H = M_CHUNK // 2


def kernel(x, w_mat, scale_x, scale_w):
    x = x.astype(jnp.bfloat16)
    w_mat = w_mat.astype(jnp.bfloat16)

    def body(x_ref, w_ref, sx_ref, sw_ref, out_ref,
             comm_f, comm_r, send_f, recv_f, send_r, recv_r,
             credit_f, credit_r, copy_sems):
        my = lax.axis_index("i")
        left = lax.rem(my + N_DEV - 1, N_DEV)
        right = lax.rem(my + 1, N_DEV)

        barrier_sem = pltpu.get_barrier_semaphore()
        for nbr in (left, right):
            pl.semaphore_signal(
                barrier_sem, inc=1,
                device_id=(nbr,), device_id_type=pl.DeviceIdType.MESH,
            )
        pl.semaphore_wait(barrier_sem, 2)

        def partial_half(c, half):
            xb = x_ref[pl.ds(c * M_CHUNK + half * H, H), :]
            return lax.dot_general(
                xb, w_ref[:, :], (((1,), (0,)), ((), ())),
                preferred_element_type=jnp.float32,
            )

        def ring_rdma(comm, ss, rs, send_sems, recv_sems, dst):
            return pltpu.make_async_remote_copy(
                src_ref=comm.at[ss],
                dst_ref=comm.at[rs],
                send_sem=send_sems.at[ss],
                recv_sem=recv_sems.at[rs],
                device_id=(dst,),
                device_id_type=pl.DeviceIdType.MESH,
            )

        def signal(sem, dst):
            pl.semaphore_signal(
                sem, inc=1, device_id=(dst,),
                device_id_type=pl.DeviceIdType.MESH,
            )

        comm_f[0, :, :] = partial_half(my, 0).astype(jnp.bfloat16)
        comm_r[0, :, :] = partial_half(my, 1).astype(jnp.bfloat16)
        for s in range(N_DEV - 1):
            ss = s % 2
            rs = (s + 1) % 2
            rdma_f = ring_rdma(comm_f, ss, rs, send_f, recv_f, right)
            rdma_r = ring_rdma(comm_r, ss, rs, send_r, recv_r, left)
            if s >= 1:
                pl.semaphore_wait(credit_f, 1)
                pl.semaphore_wait(credit_r, 1)
            rdma_f.start()
            rdma_r.start()
            cf = lax.rem(my - s - 1 + N_DEV, N_DEV)
            cr = lax.rem(my + s + 1, N_DEV)
            pf = partial_half(cf, 0)
            pr = partial_half(cr, 1)
            rdma_f.wait()
            rdma_r.wait()
            comm_f[rs, :, :] = (comm_f[rs, :, :].astype(jnp.float32) + pf
                                ).astype(jnp.bfloat16)
            comm_r[rs, :, :] = (comm_r[rs, :, :].astype(jnp.float32) + pr
                                ).astype(jnp.bfloat16)
            signal(credit_f, left)
            signal(credit_r, right)

        scale = sx_ref[0] * sw_ref[0]
        for comm in (comm_f, comm_r):
            y = comm[1, :, :].astype(jnp.float32) * scale
            yc = jnp.clip(y, -60.0, 60.0)
            comm[1, :, :] = (y * (1.0 / (1.0 + jnp.exp(-yc)))
                             ).astype(jnp.bfloat16)
        own_f = lax.rem(my + 1, N_DEV)
        own_r = lax.rem(my + N_DEV - 1, N_DEV)
        cp_f = pltpu.make_async_copy(
            comm_f.at[1], out_ref.at[pl.ds(own_f * M_CHUNK, H), :],
            copy_sems.at[0])
        cp_r = pltpu.make_async_copy(
            comm_r.at[1], out_ref.at[pl.ds(own_r * M_CHUNK + H, H), :],
            copy_sems.at[1])
        cp_f.start()
        cp_r.start()
        cp_f.wait()
        cp_r.wait()

        for t in range(N_DEV - 1):
            ss = (t + 1) % 2
            rs = t % 2
            rdma_f = ring_rdma(comm_f, ss, rs, send_f, recv_f, right)
            rdma_r = ring_rdma(comm_r, ss, rs, send_r, recv_r, left)
            pl.semaphore_wait(credit_f, 1)
            pl.semaphore_wait(credit_r, 1)
            rdma_f.start()
            rdma_r.start()
            rdma_f.wait()
            rdma_r.wait()
            gf = lax.rem(my - t + N_DEV, N_DEV)
            gr = lax.rem(my + t, N_DEV)
            cp_f = pltpu.make_async_copy(
                comm_f.at[rs], out_ref.at[pl.ds(gf * M_CHUNK, H), :],
                copy_sems.at[0])
            cp_r = pltpu.make_async_copy(
                comm_r.at[rs], out_ref.at[pl.ds(gr * M_CHUNK + H, H), :],
                copy_sems.at[1])
            cp_f.start()
            cp_r.start()
            cp_f.wait()
            cp_r.wait()
            if t < N_DEV - 2:
                signal(credit_f, left)
                signal(credit_r, right)

    return pl.pallas_call(
        body,
        out_shape=jax.ShapeDtypeStruct((M, N), jnp.bfloat16),
        in_specs=[
            pl.BlockSpec(memory_space=pltpu.VMEM),
            pl.BlockSpec(memory_space=pltpu.VMEM),
            pl.BlockSpec(memory_space=pltpu.SMEM),
            pl.BlockSpec(memory_space=pltpu.SMEM),
        ],
        out_specs=pl.BlockSpec(memory_space=pl.ANY),
        scratch_shapes=[
            pltpu.VMEM((2, H, N), jnp.bfloat16),
            pltpu.VMEM((2, H, N), jnp.bfloat16),
            pltpu.SemaphoreType.DMA((2,)),
            pltpu.SemaphoreType.DMA((2,)),
            pltpu.SemaphoreType.DMA((2,)),
            pltpu.SemaphoreType.DMA((2,)),
            pltpu.SemaphoreType.REGULAR,
            pltpu.SemaphoreType.REGULAR,
            pltpu.SemaphoreType.DMA((2,)),
        ],
        compiler_params=pltpu.CompilerParams(
            collective_id=0,
            vmem_limit_bytes=100 * 1024 * 1024,
        ),
    )(x, w_mat, scale_x, scale_w)


# device time: 751468 ns/iter; 3.6728x vs baseline; 1.0454x over previous
import jax
import jax.numpy as jnp
from jax import lax
from jax.experimental import pallas as pl
from jax.experimental.pallas import tpu as pltpu

N_DEV = 8
M = 4096
N = 8192
M_CHUNK = M // N_DEV
H = M_CHUNK // 2


def kernel(x, w_mat, scale_x, scale_w):
    x = x.astype(jnp.bfloat16)
    w_mat = w_mat.astype(jnp.bfloat16)

    def body(x_ref, w_ref, sx_ref, sw_ref, out_ref,
             comm_f, comm_r, send_f, recv_f, send_r, recv_r,
             credit_f, credit_r, copyf_sems, copyr_sems):
        my = lax.axis_index("i")

        def ring_dev(i):
            i = lax.rem(i + 2 * N_DEV, N_DEV)
            return jnp.where(i < 4, i, 11 - i)

        my_r = jnp.where(my < 4, my, 11 - my)
        left = ring_dev(my_r - 1)
        right = ring_dev(my_r + 1)

        barrier_sem = pltpu.get_barrier_semaphore()
        for nbr in (left, right):
            pl.semaphore_signal(
                barrier_sem, inc=1,
                device_id=(nbr,), device_id_type=pl.DeviceIdType.MESH,
            )
        pl.semaphore_wait(barrier_sem, 2)

        def partial_half(c, half):
            xb = x_ref[pl.ds(c * M_CHUNK + half * H, H), :]
            return lax.dot_general(
                xb, w_ref[:, :], (((1,), (0,)), ((), ())),
                preferred_element_type=jnp.float32,
            )

        def ring_rdma(comm, ss, rs, send_sems, recv_sems, dst):
            return pltpu.make_async_remote_copy(
                src_ref=comm.at[ss],
                dst_ref=comm.at[rs],
                send_sem=send_sems.at[ss],
                recv_sem=recv_sems.at[rs],
                device_id=(dst,),
                device_id_type=pl.DeviceIdType.MESH,
            )

        def signal(sem, dst):
            pl.semaphore_signal(
                sem, inc=1, device_id=(dst,),
                device_id_type=pl.DeviceIdType.MESH,
            )

        comm_f[0, :, :] = partial_half(my, 0).astype(jnp.bfloat16)
        comm_r[0, :, :] = partial_half(my, 1).astype(jnp.bfloat16)
        for s in range(N_DEV - 1):
            ss = s % 2
            rs = (s + 1) % 2
            rdma_f = ring_rdma(comm_f, ss, rs, send_f, recv_f, right)
            rdma_r = ring_rdma(comm_r, ss, rs, send_r, recv_r, left)
            if s >= 1:
                pl.semaphore_wait(credit_f, 1)
                pl.semaphore_wait(credit_r, 1)
            rdma_f.start()
            rdma_r.start()
            cf = ring_dev(my_r - s - 1)
            cr = ring_dev(my_r + s + 1)
            pf = partial_half(cf, 0)
            pr = partial_half(cr, 1)
            rdma_f.wait()
            comm_f[rs, :, :] = (comm_f[rs, :, :].astype(jnp.float32) + pf
                                ).astype(jnp.bfloat16)
            rdma_r.wait()
            comm_r[rs, :, :] = (comm_r[rs, :, :].astype(jnp.float32) + pr
                                ).astype(jnp.bfloat16)
            signal(credit_f, left)
            signal(credit_r, right)

        scale = sx_ref[0] * sw_ref[0]
        for comm in (comm_f, comm_r):
            y = comm[1, :, :].astype(jnp.float32) * scale
            yc = jnp.clip(y, -60.0, 60.0)
            comm[1, :, :] = (y * (1.0 / (1.0 + jnp.exp(-yc)))
                             ).astype(jnp.bfloat16)
        own_f = ring_dev(my_r + 1)
        own_r = ring_dev(my_r - 1)

        def out_copy(comm, slot, chunk, half, sems):
            return pltpu.make_async_copy(
                comm.at[slot],
                out_ref.at[pl.ds(chunk * M_CHUNK + half * H, H), :],
                sems.at[slot])

        prev_f = out_copy(comm_f, 1, own_f, 0, copyf_sems)
        prev_r = out_copy(comm_r, 1, own_r, 1, copyr_sems)
        prev_f.start()
        prev_r.start()

        for t in range(N_DEV - 1):
            ss = (t + 1) % 2
            rs = t % 2
            rdma_f = ring_rdma(comm_f, ss, rs, send_f, recv_f, right)
            rdma_r = ring_rdma(comm_r, ss, rs, send_r, recv_r, left)
            pl.semaphore_wait(credit_f, 1)
            pl.semaphore_wait(credit_r, 1)
            rdma_f.start()
            rdma_r.start()
            rdma_f.wait()
            rdma_r.wait()
            gf = ring_dev(my_r - t)
            gr = ring_dev(my_r + t)
            cp_f = out_copy(comm_f, rs, gf, 0, copyf_sems)
            cp_r = out_copy(comm_r, rs, gr, 1, copyr_sems)
            cp_f.start()
            cp_r.start()
            prev_f.wait()
            prev_r.wait()
            prev_f, prev_r = cp_f, cp_r
            if t < N_DEV - 2:
                signal(credit_f, left)
                signal(credit_r, right)
        prev_f.wait()
        prev_r.wait()

    return pl.pallas_call(
        body,
        out_shape=jax.ShapeDtypeStruct((M, N), jnp.bfloat16),
        in_specs=[
            pl.BlockSpec(memory_space=pltpu.VMEM),
            pl.BlockSpec(memory_space=pltpu.VMEM),
            pl.BlockSpec(memory_space=pltpu.SMEM),
            pl.BlockSpec(memory_space=pltpu.SMEM),
        ],
        out_specs=pl.BlockSpec(memory_space=pl.ANY),
        scratch_shapes=[
            pltpu.VMEM((2, H, N), jnp.bfloat16),
            pltpu.VMEM((2, H, N), jnp.bfloat16),
            pltpu.SemaphoreType.DMA((2,)),
            pltpu.SemaphoreType.DMA((2,)),
            pltpu.SemaphoreType.DMA((2,)),
            pltpu.SemaphoreType.DMA((2,)),
            pltpu.SemaphoreType.REGULAR,
            pltpu.SemaphoreType.REGULAR,
            pltpu.SemaphoreType.DMA((2,)),
            pltpu.SemaphoreType.DMA((2,)),
        ],
        compiler_params=pltpu.CompilerParams(
            collective_id=0,
            vmem_limit_bytes=100 * 1024 * 1024,
        ),
    )(x, w_mat, scale_x, scale_w)


# device time: 746140 ns/iter; 3.6990x vs baseline; 1.0071x over previous
import jax
import jax.numpy as jnp
from jax import lax
from jax.experimental import pallas as pl
from jax.experimental.pallas import tpu as pltpu

N_DEV = 8
M = 4096
N = 8192
M_CHUNK = M // N_DEV
H = M_CHUNK // 2
NSUB = 2
SH = H // NSUB


def kernel(x, w_mat, scale_x, scale_w):
    x = x.astype(jnp.bfloat16)
    w_mat = w_mat.astype(jnp.bfloat16)

    def body(x_ref, w_ref, sx_ref, sw_ref, out_ref,
             comm_f, comm_r, send_f, recv_f, send_r, recv_r,
             credit_f, credit_r, copyf_sems, copyr_sems):
        my = lax.axis_index("i")

        def ring_dev(i):
            i = lax.rem(i + 2 * N_DEV, N_DEV)
            return jnp.where(i < 4, i, 11 - i)

        my_r = jnp.where(my < 4, my, 11 - my)
        left = ring_dev(my_r - 1)
        right = ring_dev(my_r + 1)

        barrier_sem = pltpu.get_barrier_semaphore()
        for nbr in (left, right):
            pl.semaphore_signal(
                barrier_sem, inc=1,
                device_id=(nbr,), device_id_type=pl.DeviceIdType.MESH,
            )
        pl.semaphore_wait(barrier_sem, 2)

        def partial_rows(r0, nrows):
            xb = x_ref[pl.ds(r0, nrows), :]
            return lax.dot_general(
                xb, w_ref[:, :], (((1,), (0,)), ((), ())),
                preferred_element_type=jnp.float32,
            )

        def partial_half(c, half):
            return partial_rows(c * M_CHUNK + half * H, H)

        def ring_rdma(comm, ss, rs, send_sems, recv_sems, dst):
            return pltpu.make_async_remote_copy(
                src_ref=comm.at[ss],
                dst_ref=comm.at[rs],
                send_sem=send_sems.at[ss, 0],
                recv_sem=recv_sems.at[rs, 0],
                device_id=(dst,),
                device_id_type=pl.DeviceIdType.MESH,
            )

        def signal(sem, dst):
            pl.semaphore_signal(
                sem, inc=1, device_id=(dst,),
                device_id_type=pl.DeviceIdType.MESH,
            )

        def sub_rdma(comm, ss, rs, send_sems, recv_sems, dst, sub):
            return pltpu.make_async_remote_copy(
                src_ref=comm.at[ss, pl.ds(sub * SH, SH), :],
                dst_ref=comm.at[rs, pl.ds(sub * SH, SH), :],
                send_sem=send_sems.at[ss, sub],
                recv_sem=recv_sems.at[rs, sub],
                device_id=(dst,),
                device_id_type=pl.DeviceIdType.MESH,
            )

        comm_f[0, :, :] = partial_half(my, 0).astype(jnp.bfloat16)
        comm_r[0, :, :] = partial_half(my, 1).astype(jnp.bfloat16)
        for s in range(N_DEV - 1):
            ss = s % 2
            rs = (s + 1) % 2
            rdf = [sub_rdma(comm_f, ss, rs, send_f, recv_f, right, k)
                   for k in range(NSUB)]
            rdr = [sub_rdma(comm_r, ss, rs, send_r, recv_r, left, k)
                   for k in range(NSUB)]
            if s >= 1:
                pl.semaphore_wait(credit_f, 1)
                pl.semaphore_wait(credit_r, 1)
            for k in range(NSUB):
                rdf[k].start()
                rdr[k].start()
            cf = ring_dev(my_r - s - 1)
            cr = ring_dev(my_r + s + 1)
            for k in range(NSUB):
                pf = partial_rows(cf * M_CHUNK + k * SH, SH)
                pr = partial_rows(cr * M_CHUNK + H + k * SH, SH)
                rdf[k].wait()
                comm_f[rs, pl.ds(k * SH, SH), :] = (
                    comm_f[rs, pl.ds(k * SH, SH), :].astype(jnp.float32) + pf
                ).astype(jnp.bfloat16)
                rdr[k].wait()
                comm_r[rs, pl.ds(k * SH, SH), :] = (
                    comm_r[rs, pl.ds(k * SH, SH), :].astype(jnp.float32) + pr
                ).astype(jnp.bfloat16)
            signal(credit_f, left)
            signal(credit_r, right)

        scale = sx_ref[0] * sw_ref[0]
        for comm in (comm_f, comm_r):
            y = comm[1, :, :].astype(jnp.float32) * scale
            yc = jnp.clip(y, -60.0, 60.0)
            comm[1, :, :] = (y * (1.0 / (1.0 + jnp.exp(-yc)))
                             ).astype(jnp.bfloat16)
        own_f = ring_dev(my_r + 1)
        own_r = ring_dev(my_r - 1)

        def out_copy(comm, slot, chunk, half, sems):
            return pltpu.make_async_copy(
                comm.at[slot],
                out_ref.at[pl.ds(chunk * M_CHUNK + half * H, H), :],
                sems.at[slot])

        prev_f = out_copy(comm_f, 1, own_f, 0, copyf_sems)
        prev_r = out_copy(comm_r, 1, own_r, 1, copyr_sems)
        prev_f.start()
        prev_r.start()

        for t in range(N_DEV - 1):
            ss = (t + 1) % 2
            rs = t % 2
            rdma_f = ring_rdma(comm_f, ss, rs, send_f, recv_f, right)
            rdma_r = ring_rdma(comm_r, ss, rs, send_r, recv_r, left)
            pl.semaphore_wait(credit_f, 1)
            pl.semaphore_wait(credit_r, 1)
            rdma_f.start()
            rdma_r.start()
            rdma_f.wait()
            rdma_r.wait()
            gf = ring_dev(my_r - t)
            gr = ring_dev(my_r + t)
            cp_f = out_copy(comm_f, rs, gf, 0, copyf_sems)
            cp_r = out_copy(comm_r, rs, gr, 1, copyr_sems)
            cp_f.start()
            cp_r.start()
            prev_f.wait()
            prev_r.wait()
            prev_f, prev_r = cp_f, cp_r
            if t < N_DEV - 2:
                signal(credit_f, left)
                signal(credit_r, right)
        prev_f.wait()
        prev_r.wait()

    return pl.pallas_call(
        body,
        out_shape=jax.ShapeDtypeStruct((M, N), jnp.bfloat16),
        in_specs=[
            pl.BlockSpec(memory_space=pltpu.VMEM),
            pl.BlockSpec(memory_space=pltpu.VMEM),
            pl.BlockSpec(memory_space=pltpu.SMEM),
            pl.BlockSpec(memory_space=pltpu.SMEM),
        ],
        out_specs=pl.BlockSpec(memory_space=pl.ANY),
        scratch_shapes=[
            pltpu.VMEM((2, H, N), jnp.bfloat16),
            pltpu.VMEM((2, H, N), jnp.bfloat16),
            pltpu.SemaphoreType.DMA((2, NSUB)),
            pltpu.SemaphoreType.DMA((2, NSUB)),
            pltpu.SemaphoreType.DMA((2, NSUB)),
            pltpu.SemaphoreType.DMA((2, NSUB)),
            pltpu.SemaphoreType.REGULAR,
            pltpu.SemaphoreType.REGULAR,
            pltpu.SemaphoreType.DMA((2,)),
            pltpu.SemaphoreType.DMA((2,)),
        ],
        compiler_params=pltpu.CompilerParams(
            collective_id=0,
            vmem_limit_bytes=100 * 1024 * 1024,
        ),
    )(x, w_mat, scale_x, scale_w)
